# Initial kernel scaffold; baseline (speedup 1.0000x reference)
import jax
import jax.numpy as jnp
from jax.experimental import pallas as pl


def kernel(pred, gt, w1, b1, alpha, w2, b2):
    raise NotImplementedError("write your pallas kernel here")


# speedup vs baseline: 2.1810x; 2.1810x over previous
import functools

import numpy as np
import jax
import jax.numpy as jnp
from jax.experimental import pallas as pl
from jax.experimental.pallas import tpu as pltpu


_POOL_H = 112
_POOL_LANES = 128
_ROWS_PER_IMG = 3 * _POOL_H
_FEAT_DIM = 512


def _np_pool_matrix(out_size, in_size):
    i = np.arange(out_size)
    starts = (i * in_size) // out_size
    ends = ((i + 1) * in_size + out_size - 1) // out_size
    idx = np.arange(in_size)
    mask = (idx[None, :] >= starts[:, None]) & (idx[None, :] < ends[:, None])
    counts = (ends - starts).astype(np.float32)
    return mask.astype(np.float32) / counts[:, None]


@functools.lru_cache(maxsize=None)
def _combined_pool_mats(H, W):
    p112 = _np_pool_matrix(_POOL_H, 188)
    if H != 256:
        ph = _np_pool_matrix(256, H)
        pw = _np_pool_matrix(256, W)
    else:
        ph = np.eye(H, dtype=np.float32)
        pw = np.eye(W, dtype=np.float32)
    ah = p112 @ ph[35:223, :]
    aw = p112 @ pw[32:220, :]
    aw_pad = np.concatenate(
        [aw, np.zeros((_POOL_LANES - _POOL_H, W), np.float32)], axis=0)
    return (jnp.asarray(ah, jnp.bfloat16),
            jnp.asarray(aw_pad.T, jnp.bfloat16))


def _fused_idloss(pred, gt, ah, awt, w1, b1, alpha, w2, b2):
    B, threeH, W = pred.shape
    H = threeH // 3
    inv_count = 1.0 / B

    def _body(xp_ref, xg_ref, ah_ref, awt_ref, w1_ref, b1_ref, a_ref,
              w2_ref, b2_ref, o_ref):
        x = jnp.concatenate(
            [xp_ref[0].astype(jnp.bfloat16), xg_ref[0].astype(jnp.bfloat16)],
            axis=0)
        t = jnp.dot(x, awt_ref[...],
                    preferred_element_type=jnp.float32).astype(jnp.bfloat16)
        pooled = jnp.concatenate(
            [jnp.dot(ah_ref[...], t[c * H:(c + 1) * H, :],
                     preferred_element_type=jnp.float32).astype(jnp.bfloat16)
             for c in range(6)], axis=0)
        h = jnp.dot(pooled, w1_ref[...],
                    preferred_element_type=jnp.float32) + b1_ref[...]
        h = jnp.where(h > 0, h, a_ref[...] * h)
        m = jnp.concatenate(
            [jnp.mean(h[:_ROWS_PER_IMG], axis=0, keepdims=True),
             jnp.mean(h[_ROWS_PER_IMG:], axis=0, keepdims=True)], axis=0)
        f = jnp.dot(m.astype(jnp.bfloat16), w2_ref[...],
                    preferred_element_type=jnp.float32) + b2_ref[...]
        ssq = jnp.sum(f * f, axis=-1, keepdims=True)
        fn = f * jax.lax.rsqrt(jnp.maximum(ssq, 1e-12))
        d = jnp.sum(fn[0:1, :] * fn[1:2, :])
        o_ref[...] = jnp.full((1, 1, _POOL_LANES), (1.0 - d) * inv_count,
                              jnp.float32)

    out = pl.pallas_call(
        _body,
        out_shape=jax.ShapeDtypeStruct((B, 1, _POOL_LANES), jnp.float32),
        grid=(B,),
        in_specs=[
            pl.BlockSpec((1, threeH, W), lambda i: (i, 0, 0)),
            pl.BlockSpec((1, threeH, W), lambda i: (i, 0, 0)),
            pl.BlockSpec((_POOL_H, H), lambda i: (0, 0)),
            pl.BlockSpec((W, _POOL_LANES), lambda i: (0, 0)),
            pl.BlockSpec((_POOL_LANES, _FEAT_DIM), lambda i: (0, 0)),
            pl.BlockSpec((1, _FEAT_DIM), lambda i: (0, 0)),
            pl.BlockSpec((1, _FEAT_DIM), lambda i: (0, 0)),
            pl.BlockSpec((_FEAT_DIM, _FEAT_DIM), lambda i: (0, 0)),
            pl.BlockSpec((1, _FEAT_DIM), lambda i: (0, 0)),
        ],
        out_specs=pl.BlockSpec((1, 1, _POOL_LANES), lambda i: (i, 0, 0)),
        compiler_params=pltpu.CompilerParams(
            dimension_semantics=("parallel",)),
    )(pred, gt, ah, awt, w1, b1, alpha, w2, b2)
    return jnp.sum(out[:, 0, 0])


@jax.jit
def kernel(pred, gt, w1, b1, alpha, w2, b2):
    B, C, H, W = pred.shape
    gt = jax.lax.stop_gradient(gt)
    ah, awt = _combined_pool_mats(H, W)
    loss = _fused_idloss(pred.reshape(B, C * H, W), gt.reshape(B, C * H, W),
                         ah, awt, w1.astype(jnp.bfloat16), b1, alpha,
                         w2.astype(jnp.bfloat16), b2)
    return loss, jnp.float32(0.0)


# speedup vs baseline: 2.9994x; 1.3752x over previous
import functools

import numpy as np
import jax
import jax.numpy as jnp
from jax.experimental import pallas as pl
from jax.experimental.pallas import tpu as pltpu


_POOL_H = 112
_POOL_LANES = 128
_ROWS_PER_IMG = 3 * _POOL_H
_FEAT_DIM = 512


def _np_pool_matrix(out_size, in_size):
    i = np.arange(out_size)
    starts = (i * in_size) // out_size
    ends = ((i + 1) * in_size + out_size - 1) // out_size
    idx = np.arange(in_size)
    mask = (idx[None, :] >= starts[:, None]) & (idx[None, :] < ends[:, None])
    counts = (ends - starts).astype(np.float32)
    return mask.astype(np.float32) / counts[:, None]


@functools.lru_cache(maxsize=None)
def _combined_pool_mats(H, W):
    p112 = _np_pool_matrix(_POOL_H, 188)
    if H != 256:
        ph = _np_pool_matrix(256, H)
        pw = _np_pool_matrix(256, W)
    else:
        ph = np.eye(H, dtype=np.float32)
        pw = np.eye(W, dtype=np.float32)
    ah = p112 @ ph[35:223, :]
    aw = p112 @ pw[32:220, :]
    aw_pad = np.concatenate(
        [aw, np.zeros((_POOL_LANES - _POOL_H, W), np.float32)], axis=0)
    return (jnp.asarray(ah, jnp.bfloat16),
            jnp.asarray(aw_pad.T, jnp.bfloat16))


@functools.lru_cache(maxsize=None)
def _mean_selector(n_imgs):
    sel = np.zeros((n_imgs, n_imgs * _ROWS_PER_IMG), np.float32)
    for i in range(n_imgs):
        sel[i, i * _ROWS_PER_IMG:(i + 1) * _ROWS_PER_IMG] = 1.0 / _ROWS_PER_IMG
    return jnp.asarray(sel)


def _fused_idloss(pred, gt, ah, awt, sel, w1, b1, alpha, w2, b2, pairs):
    B, threeH, W = pred.shape
    H = threeH // 3
    inv_count = 1.0 / B
    n_imgs = 2 * pairs
    planes = 3 * n_imgs

    def _body(xp_ref, xg_ref, ah_ref, awt_ref, sel_ref, w1_ref, b1_ref,
              a_ref, w2_ref, b2_ref, o_ref):
        xp = xp_ref[...].reshape(pairs * threeH, W).astype(jnp.bfloat16)
        xg = xg_ref[...].reshape(pairs * threeH, W).astype(jnp.bfloat16)
        x = jnp.concatenate([xp, xg], axis=0)
        t = jnp.dot(x, awt_ref[...],
                    preferred_element_type=jnp.float32).astype(jnp.bfloat16)
        pooled = jnp.concatenate(
            [jnp.dot(ah_ref[...], t[c * H:(c + 1) * H, :],
                     preferred_element_type=jnp.float32).astype(jnp.bfloat16)
             for c in range(planes)], axis=0)
        h = jnp.dot(pooled, w1_ref[...],
                    preferred_element_type=jnp.float32) + b1_ref[...]
        h = jnp.where(h > 0, h, a_ref[...] * h)
        m = jnp.dot(sel_ref[...], h,
                    preferred_element_type=jnp.float32)
        f = jnp.dot(m.astype(jnp.bfloat16), w2_ref[...],
                    preferred_element_type=jnp.float32) + b2_ref[...]
        ssq = jnp.sum(f * f, axis=-1, keepdims=True)
        fn = f * jax.lax.rsqrt(jnp.maximum(ssq, 1e-12))
        d = jnp.sum(fn[:pairs] * fn[pairs:], axis=-1, keepdims=True)
        o_ref[...] = jnp.broadcast_to(((1.0 - d) * inv_count)[:, :, None],
                                      (pairs, 1, _POOL_LANES))

    out = pl.pallas_call(
        _body,
        out_shape=jax.ShapeDtypeStruct((B, 1, _POOL_LANES), jnp.float32),
        grid=(B // pairs,),
        in_specs=[
            pl.BlockSpec((pairs, threeH, W), lambda i: (i, 0, 0)),
            pl.BlockSpec((pairs, threeH, W), lambda i: (i, 0, 0)),
            pl.BlockSpec((_POOL_H, H), lambda i: (0, 0)),
            pl.BlockSpec((W, _POOL_LANES), lambda i: (0, 0)),
            pl.BlockSpec((n_imgs, n_imgs * _ROWS_PER_IMG), lambda i: (0, 0)),
            pl.BlockSpec((_POOL_LANES, _FEAT_DIM), lambda i: (0, 0)),
            pl.BlockSpec((1, _FEAT_DIM), lambda i: (0, 0)),
            pl.BlockSpec((1, _FEAT_DIM), lambda i: (0, 0)),
            pl.BlockSpec((_FEAT_DIM, _FEAT_DIM), lambda i: (0, 0)),
            pl.BlockSpec((1, _FEAT_DIM), lambda i: (0, 0)),
        ],
        out_specs=pl.BlockSpec((pairs, 1, _POOL_LANES), lambda i: (i, 0, 0)),
        compiler_params=pltpu.CompilerParams(
            dimension_semantics=("arbitrary",)),
    )(pred, gt, ah, awt, sel, w1, b1, alpha, w2, b2)
    return jnp.sum(out[:, 0, 0])


@jax.jit
def kernel(pred, gt, w1, b1, alpha, w2, b2):
    B, C, H, W = pred.shape
    gt = jax.lax.stop_gradient(gt)
    pairs = 4 if B % 4 == 0 else 1
    ah, awt = _combined_pool_mats(H, W)
    sel = _mean_selector(2 * pairs)
    loss = _fused_idloss(pred.reshape(B, C * H, W), gt.reshape(B, C * H, W),
                         ah, awt, sel, w1.astype(jnp.bfloat16), b1, alpha,
                         w2.astype(jnp.bfloat16), b2, pairs)
    return loss, jnp.float32(0.0)


# speedup vs baseline: 3.5359x; 1.1789x over previous
import functools

import numpy as np
import jax
import jax.numpy as jnp
from jax.experimental import pallas as pl
from jax.experimental.pallas import tpu as pltpu


_POOL_H = 112
_POOL_LANES = 128
_ROWS_PER_IMG = 3 * _POOL_H
_FEAT_DIM = 512


def _np_pool_matrix(out_size, in_size):
    i = np.arange(out_size)
    starts = (i * in_size) // out_size
    ends = ((i + 1) * in_size + out_size - 1) // out_size
    idx = np.arange(in_size)
    mask = (idx[None, :] >= starts[:, None]) & (idx[None, :] < ends[:, None])
    counts = (ends - starts).astype(np.float32)
    return mask.astype(np.float32) / counts[:, None]


@functools.lru_cache(maxsize=None)
def _combined_pool_mats(H, W):
    p112 = _np_pool_matrix(_POOL_H, 188)
    if H != 256:
        ph = _np_pool_matrix(256, H)
        pw = _np_pool_matrix(256, W)
    else:
        ph = np.eye(H, dtype=np.float32)
        pw = np.eye(W, dtype=np.float32)
    ah = p112 @ ph[35:223, :]
    aw = p112 @ pw[32:220, :]
    aw_pad = np.concatenate(
        [aw, np.zeros((_POOL_LANES - _POOL_H, W), np.float32)], axis=0)
    return (jnp.asarray(ah, jnp.bfloat16),
            jnp.asarray(aw_pad.T, jnp.bfloat16))


@functools.lru_cache(maxsize=None)
def _mean_selector(n_imgs):
    sel = np.zeros((n_imgs, n_imgs * _ROWS_PER_IMG), np.float32)
    for i in range(n_imgs):
        sel[i, i * _ROWS_PER_IMG:(i + 1) * _ROWS_PER_IMG] = 1.0 / _ROWS_PER_IMG
    return jnp.asarray(sel)


def _fused_idloss(pred, gt, ah, awt, sel, w1, b1, alpha, w2, b2, pairs):
    B, threeH, W = pred.shape
    H = threeH // 3
    inv_count = 1.0 / B
    n_imgs = 2 * pairs
    planes = 3 * n_imgs

    def _body(xp_ref, xg_ref, ah_ref, awt_ref, sel_ref, w1_ref, b1_ref,
              a_ref, w2_ref, b2_ref, o_ref):
        @pl.when(pl.program_id(0) == 0)
        def _():
            o_ref[...] = jnp.zeros_like(o_ref)

        xp = xp_ref[...].reshape(pairs * threeH, W).astype(jnp.bfloat16)
        xg = xg_ref[...].reshape(pairs * threeH, W).astype(jnp.bfloat16)
        tp = jnp.dot(xp, awt_ref[...],
                     preferred_element_type=jnp.float32).astype(jnp.bfloat16)
        tg = jnp.dot(xg, awt_ref[...],
                     preferred_element_type=jnp.float32).astype(jnp.bfloat16)
        pooled = jnp.concatenate(
            [jnp.dot(ah_ref[...], t[c * H:(c + 1) * H, :],
                     preferred_element_type=jnp.float32).astype(jnp.bfloat16)
             for t in (tp, tg) for c in range(planes // 2)],
            axis=0)
        h = jnp.dot(pooled, w1_ref[...],
                    preferred_element_type=jnp.float32) + b1_ref[...]
        h = jnp.where(h > 0, h, a_ref[...] * h)
        m = jnp.dot(sel_ref[...], h,
                    preferred_element_type=jnp.float32)
        f = jnp.dot(m.astype(jnp.bfloat16), w2_ref[...],
                    preferred_element_type=jnp.float32) + b2_ref[...]
        ssq = jnp.sum(f * f, axis=-1, keepdims=True)
        fn = f * jax.lax.rsqrt(jnp.maximum(ssq, 1e-12))
        d = jnp.sum(fn[:pairs] * fn[pairs:], axis=-1, keepdims=True)
        o_ref[...] += jnp.sum((1.0 - d) * inv_count, keepdims=True)

    out = pl.pallas_call(
        _body,
        out_shape=jax.ShapeDtypeStruct((1, 1), jnp.float32),
        grid=(B // pairs,),
        in_specs=[
            pl.BlockSpec((pairs, threeH, W), lambda i: (i, 0, 0)),
            pl.BlockSpec((pairs, threeH, W), lambda i: (i, 0, 0)),
            pl.BlockSpec((_POOL_H, H), lambda i: (0, 0)),
            pl.BlockSpec((W, _POOL_LANES), lambda i: (0, 0)),
            pl.BlockSpec((n_imgs, n_imgs * _ROWS_PER_IMG), lambda i: (0, 0)),
            pl.BlockSpec((_POOL_LANES, _FEAT_DIM), lambda i: (0, 0)),
            pl.BlockSpec((1, _FEAT_DIM), lambda i: (0, 0)),
            pl.BlockSpec((1, _FEAT_DIM), lambda i: (0, 0)),
            pl.BlockSpec((_FEAT_DIM, _FEAT_DIM), lambda i: (0, 0)),
            pl.BlockSpec((1, _FEAT_DIM), lambda i: (0, 0)),
        ],
        out_specs=pl.BlockSpec((1, 1), lambda i: (0, 0)),
        compiler_params=pltpu.CompilerParams(
            dimension_semantics=("arbitrary",)),
    )(pred, gt, ah, awt, sel, w1, b1, alpha, w2, b2)
    return out[0, 0]


@jax.jit
def kernel(pred, gt, w1, b1, alpha, w2, b2):
    B, C, H, W = pred.shape
    gt = jax.lax.stop_gradient(gt)
    pairs = 4 if B % 4 == 0 else 1
    ah, awt = _combined_pool_mats(H, W)
    sel = _mean_selector(2 * pairs)
    loss = _fused_idloss(pred.reshape(B, C * H, W), gt.reshape(B, C * H, W),
                         ah, awt, sel, w1.astype(jnp.bfloat16), b1, alpha,
                         w2.astype(jnp.bfloat16), b2, pairs)
    return loss, jnp.float32(0.0)


# speedup vs baseline: 3.6416x; 1.0299x over previous
import functools

import numpy as np
import jax
import jax.numpy as jnp
from jax.experimental import pallas as pl
from jax.experimental.pallas import tpu as pltpu


_POOL_H = 112
_POOL_LANES = 128
_ROWS_PER_IMG = 3 * _POOL_H
_FEAT_DIM = 512


def _np_pool_matrix(out_size, in_size):
    i = np.arange(out_size)
    starts = (i * in_size) // out_size
    ends = ((i + 1) * in_size + out_size - 1) // out_size
    idx = np.arange(in_size)
    mask = (idx[None, :] >= starts[:, None]) & (idx[None, :] < ends[:, None])
    counts = (ends - starts).astype(np.float32)
    return mask.astype(np.float32) / counts[:, None]


@functools.lru_cache(maxsize=None)
def _combined_pool_mats(H, W):
    p112 = _np_pool_matrix(_POOL_H, 188)
    if H != 256:
        ph = _np_pool_matrix(256, H)
        pw = _np_pool_matrix(256, W)
    else:
        ph = np.eye(H, dtype=np.float32)
        pw = np.eye(W, dtype=np.float32)
    ah = p112 @ ph[35:223, :]
    aw = p112 @ pw[32:220, :]
    aw_pad = np.concatenate(
        [aw, np.zeros((_POOL_LANES - _POOL_H, W), np.float32)], axis=0)
    return (jnp.asarray(ah, jnp.bfloat16),
            jnp.asarray(aw_pad.T, jnp.bfloat16))


@functools.lru_cache(maxsize=None)
def _mean_selector(n_imgs):
    sel = np.zeros((n_imgs, n_imgs * _ROWS_PER_IMG), np.float32)
    for i in range(n_imgs):
        sel[i, i * _ROWS_PER_IMG:(i + 1) * _ROWS_PER_IMG] = 1.0 / _ROWS_PER_IMG
    return jnp.asarray(sel)


def _fused_idloss(pred, gt, ah, awt, sel, w1, b1, alpha, w2, b2, pairs):
    B, threeH, W = pred.shape
    H = threeH // 3
    inv_count = 1.0 / B
    n_imgs = 2 * pairs
    planes = 3 * n_imgs

    def _body(xp_ref, xg_ref, ah_ref, awt_ref, sel_ref, w1_ref, b1_ref,
              a_ref, w2_ref, b2_ref, o_ref):
        @pl.when(pl.program_id(0) == 0)
        def _():
            o_ref[...] = jnp.zeros_like(o_ref)

        xp = xp_ref[...].reshape(pairs * threeH, W).astype(jnp.bfloat16)
        xg = xg_ref[...].reshape(pairs * threeH, W).astype(jnp.bfloat16)
        tp = jnp.dot(xp, awt_ref[...],
                     preferred_element_type=jnp.float32).astype(jnp.bfloat16)
        tg = jnp.dot(xg, awt_ref[...],
                     preferred_element_type=jnp.float32).astype(jnp.bfloat16)
        pooled = jnp.concatenate(
            [jnp.dot(ah_ref[...], t[c * H:(c + 1) * H, :],
                     preferred_element_type=jnp.float32).astype(jnp.bfloat16)
             for t in (tp, tg) for c in range(planes // 2)],
            axis=0)
        h = jnp.dot(pooled, w1_ref[...],
                    preferred_element_type=jnp.float32) + b1_ref[...]
        h = jnp.where(h > 0, h, a_ref[...] * h)
        m = jnp.dot(sel_ref[...], h,
                    preferred_element_type=jnp.float32)
        f = jnp.dot(m.astype(jnp.bfloat16), w2_ref[...],
                    preferred_element_type=jnp.float32) + b2_ref[...]
        ssq = jnp.sum(f * f, axis=-1, keepdims=True)
        fn = f * jax.lax.rsqrt(jnp.maximum(ssq, 1e-12))
        d = jnp.sum(fn[:pairs] * fn[pairs:], axis=-1, keepdims=True)
        o_ref[...] += jnp.sum((1.0 - d) * inv_count, keepdims=True)

    out = pl.pallas_call(
        _body,
        out_shape=jax.ShapeDtypeStruct((1, 1), jnp.float32),
        grid=(B // pairs,),
        in_specs=[
            pl.BlockSpec((pairs, threeH, W), lambda i: (i, 0, 0)),
            pl.BlockSpec((pairs, threeH, W), lambda i: (i, 0, 0)),
            pl.BlockSpec((_POOL_H, H), lambda i: (0, 0)),
            pl.BlockSpec((W, _POOL_LANES), lambda i: (0, 0)),
            pl.BlockSpec((n_imgs, n_imgs * _ROWS_PER_IMG), lambda i: (0, 0)),
            pl.BlockSpec((_POOL_LANES, _FEAT_DIM), lambda i: (0, 0)),
            pl.BlockSpec((1, _FEAT_DIM), lambda i: (0, 0)),
            pl.BlockSpec((1, _FEAT_DIM), lambda i: (0, 0)),
            pl.BlockSpec((_FEAT_DIM, _FEAT_DIM), lambda i: (0, 0)),
            pl.BlockSpec((1, _FEAT_DIM), lambda i: (0, 0)),
        ],
        out_specs=pl.BlockSpec((1, 1), lambda i: (0, 0)),
        compiler_params=pltpu.CompilerParams(
            dimension_semantics=("arbitrary",)),
    )(pred, gt, ah, awt, sel, w1, b1, alpha, w2, b2)
    return out[0, 0]


@jax.jit
def kernel(pred, gt, w1, b1, alpha, w2, b2):
    B, C, H, W = pred.shape
    gt = jax.lax.stop_gradient(gt)
    pairs = 8 if B % 8 == 0 else (4 if B % 4 == 0 else 1)
    ah, awt = _combined_pool_mats(H, W)
    sel = _mean_selector(2 * pairs)
    loss = _fused_idloss(pred.reshape(B, C * H, W), gt.reshape(B, C * H, W),
                         ah, awt, sel, w1.astype(jnp.bfloat16), b1, alpha,
                         w2.astype(jnp.bfloat16), b2, pairs)
    return loss, jnp.float32(0.0)


# speedup vs baseline: 4.0939x; 1.1242x over previous
import functools

import numpy as np
import jax
import jax.numpy as jnp
from jax.experimental import pallas as pl
from jax.experimental.pallas import tpu as pltpu


_POOL_H = 112
_POOL_LANES = 128
_ROWS_PER_IMG = 3 * _POOL_H
_FEAT_DIM = 512


def _np_pool_matrix(out_size, in_size):
    i = np.arange(out_size)
    starts = (i * in_size) // out_size
    ends = ((i + 1) * in_size + out_size - 1) // out_size
    idx = np.arange(in_size)
    mask = (idx[None, :] >= starts[:, None]) & (idx[None, :] < ends[:, None])
    counts = (ends - starts).astype(np.float32)
    return mask.astype(np.float32) / counts[:, None]


@functools.lru_cache(maxsize=None)
def _combined_pool_mats(H, W):
    p112 = _np_pool_matrix(_POOL_H, 188)
    if H != 256:
        ph = _np_pool_matrix(256, H)
        pw = _np_pool_matrix(256, W)
    else:
        ph = np.eye(H, dtype=np.float32)
        pw = np.eye(W, dtype=np.float32)
    ah = p112 @ ph[35:223, :]
    aw = p112 @ pw[32:220, :]
    aw_pad = np.concatenate(
        [aw, np.zeros((_POOL_LANES - _POOL_H, W), np.float32)], axis=0)
    return (jnp.asarray(ah, jnp.bfloat16),
            jnp.asarray(aw_pad.T, jnp.bfloat16))


@functools.lru_cache(maxsize=None)
def _mean_selector(n_imgs):
    sel = np.zeros((n_imgs, n_imgs * _ROWS_PER_IMG), np.float32)
    for i in range(n_imgs):
        sel[i, i * _ROWS_PER_IMG:(i + 1) * _ROWS_PER_IMG] = 1.0 / _ROWS_PER_IMG
    return jnp.asarray(sel)


def _fused_idloss(pred, gt, ah, awt, sel, w1, b1, alpha, w2, b2, pairs):
    B, C, H, W = pred.shape
    threeH = C * H
    inv_count = 1.0 / B
    n_imgs = 2 * pairs
    planes = 3 * n_imgs

    def _body(xp_ref, xg_ref, ah_ref, awt_ref, sel_ref, w1_ref, b1_ref,
              a_ref, w2_ref, b2_ref, o_ref):
        @pl.when(pl.program_id(0) == 0)
        def _():
            o_ref[...] = jnp.zeros_like(o_ref)

        xp = xp_ref[...].reshape(pairs * threeH, W).astype(jnp.bfloat16)
        xg = xg_ref[...].reshape(pairs * threeH, W).astype(jnp.bfloat16)
        w1 = w1_ref[...].astype(jnp.bfloat16)
        w2 = w2_ref[...].astype(jnp.bfloat16)
        tp = jnp.dot(xp, awt_ref[...],
                     preferred_element_type=jnp.float32).astype(jnp.bfloat16)
        tg = jnp.dot(xg, awt_ref[...],
                     preferred_element_type=jnp.float32).astype(jnp.bfloat16)
        pooled = jnp.concatenate(
            [jnp.dot(ah_ref[...], t[c * H:(c + 1) * H, :],
                     preferred_element_type=jnp.float32).astype(jnp.bfloat16)
             for t in (tp, tg) for c in range(planes // 2)],
            axis=0)
        h = jnp.dot(pooled, w1,
                    preferred_element_type=jnp.float32) + b1_ref[...]
        h = jnp.where(h > 0, h, a_ref[...] * h)
        m = jnp.dot(sel_ref[...], h,
                    preferred_element_type=jnp.float32)
        f = jnp.dot(m.astype(jnp.bfloat16), w2,
                    preferred_element_type=jnp.float32) + b2_ref[...]
        ssq = jnp.sum(f * f, axis=-1, keepdims=True)
        fn = f * jax.lax.rsqrt(jnp.maximum(ssq, 1e-12))
        d = jnp.sum(fn[:pairs] * fn[pairs:], axis=-1, keepdims=True)
        o_ref[...] += jnp.sum((1.0 - d) * inv_count, keepdims=True)

    out = pl.pallas_call(
        _body,
        out_shape=jax.ShapeDtypeStruct((1, 1), jnp.float32),
        grid=(B // pairs,),
        in_specs=[
            pl.BlockSpec((pairs, C, H, W), lambda i: (i, 0, 0, 0)),
            pl.BlockSpec((pairs, C, H, W), lambda i: (i, 0, 0, 0)),
            pl.BlockSpec((_POOL_H, H), lambda i: (0, 0)),
            pl.BlockSpec((W, _POOL_LANES), lambda i: (0, 0)),
            pl.BlockSpec((n_imgs, n_imgs * _ROWS_PER_IMG), lambda i: (0, 0)),
            pl.BlockSpec((_POOL_LANES, _FEAT_DIM), lambda i: (0, 0)),
            pl.BlockSpec((1, _FEAT_DIM), lambda i: (0, 0)),
            pl.BlockSpec((1, _FEAT_DIM), lambda i: (0, 0)),
            pl.BlockSpec((_FEAT_DIM, _FEAT_DIM), lambda i: (0, 0)),
            pl.BlockSpec((1, _FEAT_DIM), lambda i: (0, 0)),
        ],
        out_specs=pl.BlockSpec((1, 1), lambda i: (0, 0)),
        compiler_params=pltpu.CompilerParams(
            dimension_semantics=("arbitrary",)),
    )(pred, gt, ah, awt, sel, w1, b1, alpha, w2, b2)
    return out[0, 0]


@jax.jit
def kernel(pred, gt, w1, b1, alpha, w2, b2):
    B, C, H, W = pred.shape
    gt = jax.lax.stop_gradient(gt)
    pairs = 8 if B % 8 == 0 else (4 if B % 4 == 0 else 1)
    ah, awt = _combined_pool_mats(H, W)
    sel = _mean_selector(2 * pairs)
    loss = _fused_idloss(pred, gt, ah, awt, sel, w1, b1, alpha, w2, b2, pairs)
    return loss, jnp.float32(0.0)
